# baseline (speedup 1.0000x reference)
import functools
import jax
import jax.numpy as jnp
from jax import lax
from jax.experimental import pallas as pl
from jax.experimental.pallas import tpu as pltpu
from jax.experimental.pallas import tpu_sc as plsc

_ALPHA = 0.75
_GAMMA = 2.0
_DICE_WEIGHT = 0.5
_SMOOTH = 1e-06

_ROWS = 4096
_COLS = 1024
_N = _ROWS * _COLS
_K = _N // 4
_NCHUNK = 16

_NW = 32
_PER_W = _N // _NW
_SCROWS = 32
_SCCHUNK = _SCROWS * 512
_NSCCHUNK = _PER_W // _SCCHUNK
_L = 16

_HI_BINS = 1 << 16
_LO_BINS = 1 << 15


def _focal_kernel(pred_ref, target_ref, bits_ref, dice_ref):
    i = pl.program_id(0)
    p = pred_ref[...]
    t = target_ref[...]
    bce = jnp.maximum(p, 0.0) - p * t + jnp.log1p(jnp.exp(-jnp.abs(p)))
    prob = jax.nn.sigmoid(p)
    om = jnp.where(t > 0.5, 1.0 - prob, prob)
    alpha_t = t * _ALPHA + (1.0 - t) * (1.0 - _ALPHA)
    focal = alpha_t * (om * om) * bce
    bits_ref[...] = lax.bitcast_convert_type(focal, jnp.int32)

    s_prob = jnp.sum(prob)
    s_tgt = jnp.sum(t)
    s_int = jnp.sum(prob * t)

    @pl.when(i == 0)
    def _():
        dice_ref[0] = s_prob
        dice_ref[1] = s_tgt
        dice_ref[2] = s_int

    @pl.when(i > 0)
    def _():
        dice_ref[0] += s_prob
        dice_ref[1] += s_tgt
        dice_ref[2] += s_int


def _tc_focal(pred4, target4):
    return pl.pallas_call(
        _focal_kernel,
        grid=(_NCHUNK,),
        in_specs=[
            pl.BlockSpec((16 // _NCHUNK, 1, 512, 512), lambda i: (i, 0, 0, 0)),
            pl.BlockSpec((16 // _NCHUNK, 1, 512, 512), lambda i: (i, 0, 0, 0)),
        ],
        out_specs=[
            pl.BlockSpec((16 // _NCHUNK, 1, 512, 512), lambda i: (i, 0, 0, 0)),
            pl.BlockSpec(memory_space=pltpu.SMEM),
        ],
        out_shape=[
            jax.ShapeDtypeStruct((16, 1, 512, 512), jnp.int32),
            jax.ShapeDtypeStruct((3,), jnp.float32),
        ],
    )(pred4, target4)


def _sc_mesh():
    return plsc.VectorSubcoreMesh(core_axis_name="c", subcore_axis_name="s")


_SC_PARAMS = pltpu.CompilerParams(needs_layout_passes=False)


def _chunk_src(bits_hbm, wid, c):
    img = wid // 2
    r0 = (wid % 2) * 256 + c * _SCROWS
    return bits_hbm.at[img, 0, pl.ds(r0, _SCROWS), :]


def _sc_hist_a(bits4):
    @functools.partial(
        pl.kernel,
        mesh=_sc_mesh(),
        compiler_params=_SC_PARAMS,
        out_type=jax.ShapeDtypeStruct((_NW, _HI_BINS), jnp.int32),
        scratch_types=[
            pltpu.VMEM((_SCROWS, 512), jnp.int32),
            pltpu.VMEM((_SCROWS, 512), jnp.int32),
            pltpu.VMEM((_HI_BINS,), jnp.int32),
            pltpu.SemaphoreType.DMA((2,)),
        ],
    )
    def k(bits_hbm, hist_hbm, buf0, buf1, hist, sem):
        wid = lax.axis_index("s") * 2 + lax.axis_index("c")
        zeros = jnp.zeros((_L,), jnp.int32)
        ones = jnp.ones((_L,), jnp.int32)
        bufs = [buf0, buf1]

        copies = [None, None]
        copies[0] = pltpu.async_copy(_chunk_src(bits_hbm, wid, 0), buf0,
                                     sem.at[0])

        @plsc.parallel_loop(0, _HI_BINS, step=_L, unroll=16)
        def _(i):
            hist[pl.ds(i, _L)] = zeros

        for c in range(_NSCCHUNK):
            b = c % 2
            if c + 1 < _NSCCHUNK:
                nb = (c + 1) % 2
                copies[nb] = pltpu.async_copy(
                    _chunk_src(bits_hbm, wid, c + 1), bufs[nb], sem.at[nb])
            copies[b].wait()
            cur = bufs[b]

            @plsc.parallel_loop(0, _SCCHUNK, step=_L, unroll=16)
            def _(i):
                v = cur[i >> 9, pl.ds(i & 511, _L)]
                idx = lax.shift_right_logical(v, 15)
                plsc.addupdate_scatter(hist, [idx], ones)

        pltpu.sync_copy(hist, hist_hbm.at[wid])

    return k(bits4)


def _tc_select_bin(histA):
    def k(h_ref, out_ref):
        h = jnp.sum(h_ref[...], axis=0).reshape(512, 128)
        r_iota = lax.broadcasted_iota(jnp.int32, (512, 128), 0)
        c_iota = lax.broadcasted_iota(jnp.int32, (512, 128), 1)
        bin_idx = r_iota * 128 + c_iota

        def cnt_ge(m):
            return jnp.sum(jnp.where(bin_idx >= m, h, 0))

        def body(_, carry):
            lo, hi = carry
            mid = lo + (hi - lo + 1) // 2
            big = cnt_ge(mid) >= _K
            return jnp.where(big, mid, lo), jnp.where(big, hi, mid - 1)

        B, _ = lax.fori_loop(0, 17, body, (jnp.int32(0), jnp.int32(_HI_BINS - 1)))
        out_ref[0] = B
        out_ref[1] = cnt_ge(B + 1)

    return pl.pallas_call(
        k,
        out_specs=pl.BlockSpec(memory_space=pltpu.SMEM),
        out_shape=jax.ShapeDtypeStruct((2,), jnp.int32),
    )(histA)


def _sc_hist_b(bits3, b_vec):
    @functools.partial(
        pl.kernel,
        mesh=_sc_mesh(),
        compiler_params=_SC_PARAMS,
        out_type=[
            jax.ShapeDtypeStruct((_NW, _LO_BINS), jnp.int32),
            jax.ShapeDtypeStruct((_NW, _L), jnp.float32),
        ],
        scratch_types=[
            pltpu.VMEM((_SCROWS, 512), jnp.int32),
            pltpu.VMEM((_SCROWS, 512), jnp.int32),
            pltpu.VMEM((_LO_BINS,), jnp.int32),
            pltpu.VMEM((_L,), jnp.int32),
            pltpu.VMEM((_L,), jnp.float32),
            pltpu.SemaphoreType.DMA((2,)),
        ],
    )
    def k(bits_hbm, b_hbm, hist_hbm, sum_hbm, buf0, buf1, hist, bbuf, acc, sem):
        wid = lax.axis_index("s") * 2 + lax.axis_index("c")
        zeros = jnp.zeros((_L,), jnp.int32)
        ones = jnp.ones((_L,), jnp.int32)
        fz = jnp.zeros((_L,), jnp.float32)
        bufs = [buf0, buf1]

        copies = [None, None]
        copies[0] = pltpu.async_copy(_chunk_src(bits_hbm, wid, 0), buf0,
                                     sem.at[0])

        pltpu.sync_copy(b_hbm, bbuf)
        blo = lax.shift_left(bbuf[...], 15)
        bhi = blo + _LO_BINS

        @plsc.parallel_loop(0, _LO_BINS, step=_L, unroll=16)
        def _(i):
            hist[pl.ds(i, _L)] = zeros

        total = fz
        for c in range(_NSCCHUNK):
            b = c % 2
            if c + 1 < _NSCCHUNK:
                nb = (c + 1) % 2
                copies[nb] = pltpu.async_copy(
                    _chunk_src(bits_hbm, wid, c + 1), bufs[nb], sem.at[nb])
            copies[b].wait()
            cur = bufs[b]

            @plsc.parallel_loop(0, _SCCHUNK, step=_L, unroll=16, carry=fz)
            def sub(i, s):
                v = cur[i >> 9, pl.ds(i & 511, _L)]
                m_a = v >= bhi
                m_b = (v >= blo) & (~m_a)
                low = v - blo
                plsc.addupdate_scatter(hist, [low], ones, mask=m_b)
                f = plsc.bitcast(v, jnp.float32)
                return s + jnp.where(m_a, f, fz)

            total = total + sub

        pltpu.sync_copy(hist, hist_hbm.at[wid])
        acc[...] = total
        pltpu.sync_copy(acc, sum_hbm.at[wid])

    return k(bits3, b_vec)


def _tc_final(histB, sums, sel, dice):
    def k(h_ref, s_ref, sel_ref, dice_ref, out_ref):
        B = sel_ref[0]
        count_above = sel_ref[1]
        r = _K - count_above

        h = jnp.sum(h_ref[...], axis=0).reshape(256, 128)
        r_iota = lax.broadcasted_iota(jnp.int32, (256, 128), 0)
        c_iota = lax.broadcasted_iota(jnp.int32, (256, 128), 1)
        j_idx = r_iota * 128 + c_iota

        def cnt_ge(m):
            return jnp.sum(jnp.where(j_idx >= m, h, 0))

        def body(_, carry):
            lo, hi = carry
            mid = lo + (hi - lo + 1) // 2
            big = cnt_ge(mid) >= r
            return jnp.where(big, mid, lo), jnp.where(big, hi, mid - 1)

        jstar, _ = lax.fori_loop(0, 16, body,
                                 (jnp.int32(0), jnp.int32(_LO_BINS - 1)))
        cnt_gt_in_b = cnt_ge(jstar + 1)

        vals = lax.bitcast_convert_type(B * _LO_BINS + j_idx, jnp.float32)
        sum_gt_in_b = jnp.sum(
            jnp.where(j_idx > jstar, vals * h.astype(jnp.float32), 0.0))
        t_val = lax.bitcast_convert_type(B * _LO_BINS + jstar, jnp.float32)

        sum_above = jnp.sum(s_ref[...])
        n_t = (_K - count_above - cnt_gt_in_b).astype(jnp.float32)
        focal_loss = (sum_above + sum_gt_in_b + t_val * n_t) / _K

        dice_loss = 1.0 - (2.0 * dice_ref[2] + _SMOOTH) / (
            dice_ref[0] + dice_ref[1] + _SMOOTH)
        out_ref[0] = _DICE_WEIGHT * dice_loss + (1.0 - _DICE_WEIGHT) * focal_loss

    return pl.pallas_call(
        k,
        in_specs=[
            pl.BlockSpec((_NW, _LO_BINS), lambda: (0, 0)),
            pl.BlockSpec((_NW, _L), lambda: (0, 0)),
            pl.BlockSpec(memory_space=pltpu.SMEM),
            pl.BlockSpec(memory_space=pltpu.SMEM),
        ],
        out_specs=pl.BlockSpec(memory_space=pltpu.SMEM),
        out_shape=jax.ShapeDtypeStruct((1,), jnp.float32),
    )(histB, sums, sel, dice)


def kernel(pred, target):
    bits, dice = _tc_focal(pred, target)
    histA = _sc_hist_a(bits)
    sel = _tc_select_bin(histA)
    b_vec = jnp.broadcast_to(sel[0:1], (_L,))
    histB, sums = _sc_hist_b(bits, b_vec)
    return _tc_final(histB, sums, sel, dice)[0]


# speedup vs baseline: 1.7524x; 1.7524x over previous
import functools
import jax
import jax.numpy as jnp
from jax import lax
from jax.experimental import pallas as pl
from jax.experimental.pallas import tpu as pltpu
from jax.experimental.pallas import tpu_sc as plsc

_ALPHA = 0.75
_GAMMA = 2.0
_DICE_WEIGHT = 0.5
_SMOOTH = 1e-06

_ROWS = 4096
_COLS = 1024
_N = _ROWS * _COLS
_K = _N // 4
_NCHUNK = 16

_NW = 32
_PER_W = _N // _NW
_SCROWS = 32
_SCCHUNK = _SCROWS * 512
_NSCCHUNK = _PER_W // _SCCHUNK
_L = 16

_HI_BINS = 1 << 16
_LO_BINS = 1 << 15


def _focal_kernel(pred_ref, target_ref, bits_ref, dice_ref):
    i = pl.program_id(0)
    p = pred_ref[...]
    t = target_ref[...]
    bce = jnp.maximum(p, 0.0) - p * t + jnp.log1p(jnp.exp(-jnp.abs(p)))
    prob = jax.nn.sigmoid(p)
    om = jnp.where(t > 0.5, 1.0 - prob, prob)
    alpha_t = t * _ALPHA + (1.0 - t) * (1.0 - _ALPHA)
    focal = alpha_t * (om * om) * bce
    bits_ref[...] = lax.bitcast_convert_type(focal, jnp.int32)

    s_prob = jnp.sum(prob)
    s_tgt = jnp.sum(t)
    s_int = jnp.sum(prob * t)

    @pl.when(i == 0)
    def _():
        dice_ref[0] = s_prob
        dice_ref[1] = s_tgt
        dice_ref[2] = s_int

    @pl.when(i > 0)
    def _():
        dice_ref[0] += s_prob
        dice_ref[1] += s_tgt
        dice_ref[2] += s_int


def _tc_focal(pred4, target4):
    return pl.pallas_call(
        _focal_kernel,
        grid=(_NCHUNK,),
        in_specs=[
            pl.BlockSpec((16 // _NCHUNK, 1, 512, 512), lambda i: (i, 0, 0, 0)),
            pl.BlockSpec((16 // _NCHUNK, 1, 512, 512), lambda i: (i, 0, 0, 0)),
        ],
        out_specs=[
            pl.BlockSpec((16 // _NCHUNK, 1, 512, 512), lambda i: (i, 0, 0, 0)),
            pl.BlockSpec(memory_space=pltpu.SMEM),
        ],
        out_shape=[
            jax.ShapeDtypeStruct((16, 1, 512, 512), jnp.int32),
            jax.ShapeDtypeStruct((3,), jnp.float32),
        ],
    )(pred4, target4)


def _sc_mesh():
    return plsc.VectorSubcoreMesh(core_axis_name="c", subcore_axis_name="s")


_SC_PARAMS = pltpu.CompilerParams(needs_layout_passes=False)


def _chunk_src(bits_hbm, wid, c):
    img = wid // 2
    r0 = (wid % 2) * 256 + c * _SCROWS
    return bits_hbm.at[img, 0, pl.ds(r0, _SCROWS), :]


def _sc_hist_a(bits4):
    @functools.partial(
        pl.kernel,
        mesh=_sc_mesh(),
        compiler_params=_SC_PARAMS,
        out_type=jax.ShapeDtypeStruct((_NW, _HI_BINS), jnp.int32),
        scratch_types=[
            pltpu.VMEM((_SCROWS, 512), jnp.int32),
            pltpu.VMEM((_SCROWS, 512), jnp.int32),
            pltpu.VMEM((_HI_BINS,), jnp.int32),
            pltpu.SemaphoreType.DMA((2,)),
        ],
    )
    def k(bits_hbm, hist_hbm, buf0, buf1, hist, sem):
        wid = lax.axis_index("s") * 2 + lax.axis_index("c")
        zeros = jnp.zeros((_L,), jnp.int32)
        ones = jnp.ones((_L,), jnp.int32)
        bufs = [buf0, buf1]

        copies = [None, None]
        copies[0] = pltpu.async_copy(_chunk_src(bits_hbm, wid, 0), buf0,
                                     sem.at[0])

        @plsc.parallel_loop(0, _HI_BINS, step=_L, unroll=8)
        def _(i):
            hist[pl.ds(i, _L)] = zeros

        for c in range(_NSCCHUNK):
            b = c % 2
            if c + 1 < _NSCCHUNK:
                nb = (c + 1) % 2
                copies[nb] = pltpu.async_copy(
                    _chunk_src(bits_hbm, wid, c + 1), bufs[nb], sem.at[nb])
            copies[b].wait()
            cur = bufs[b]

            @plsc.parallel_loop(0, _SCCHUNK, step=_L, unroll=16)
            def _(i):
                v = cur[i >> 9, pl.ds(i & 511, _L)]
                idx = lax.shift_right_logical(v, 15)
                plsc.addupdate_scatter(hist, [idx], ones)

        pltpu.sync_copy(hist, hist_hbm.at[wid])

    return k(bits4)


def _tc_select_bin(histA):
    def k(h_ref, out_ref):
        h = jnp.sum(h_ref[...], axis=0).reshape(512, 128)
        r_iota = lax.broadcasted_iota(jnp.int32, (512, 128), 0)
        c_iota = lax.broadcasted_iota(jnp.int32, (512, 128), 1)
        bin_idx = r_iota * 128 + c_iota

        def cnt_ge(m):
            return jnp.sum(jnp.where(bin_idx >= m, h, 0))

        def body(_, carry):
            lo, hi = carry
            mid = lo + (hi - lo + 1) // 2
            big = cnt_ge(mid) >= _K
            return jnp.where(big, mid, lo), jnp.where(big, hi, mid - 1)

        B, _ = lax.fori_loop(0, 17, body, (jnp.int32(0), jnp.int32(_HI_BINS - 1)))
        out_ref[0] = B
        out_ref[1] = cnt_ge(B + 1)

    return pl.pallas_call(
        k,
        out_specs=pl.BlockSpec(memory_space=pltpu.SMEM),
        out_shape=jax.ShapeDtypeStruct((2,), jnp.int32),
    )(histA)


def _sc_hist_b(bits3, b_vec):
    @functools.partial(
        pl.kernel,
        mesh=_sc_mesh(),
        compiler_params=_SC_PARAMS,
        out_type=[
            jax.ShapeDtypeStruct((_NW, _LO_BINS), jnp.int32),
            jax.ShapeDtypeStruct((_NW, _L), jnp.float32),
        ],
        scratch_types=[
            pltpu.VMEM((_SCROWS, 512), jnp.int32),
            pltpu.VMEM((_SCROWS, 512), jnp.int32),
            pltpu.VMEM((_LO_BINS,), jnp.int32),
            pltpu.VMEM((_L,), jnp.int32),
            pltpu.VMEM((_L,), jnp.float32),
            pltpu.SemaphoreType.DMA((2,)),
        ],
    )
    def k(bits_hbm, b_hbm, hist_hbm, sum_hbm, buf0, buf1, hist, bbuf, acc, sem):
        wid = lax.axis_index("s") * 2 + lax.axis_index("c")
        zeros = jnp.zeros((_L,), jnp.int32)
        ones = jnp.ones((_L,), jnp.int32)
        fz = jnp.zeros((_L,), jnp.float32)
        bufs = [buf0, buf1]

        copies = [None, None]
        copies[0] = pltpu.async_copy(_chunk_src(bits_hbm, wid, 0), buf0,
                                     sem.at[0])

        pltpu.sync_copy(b_hbm, bbuf)
        bv = bbuf[...]

        @plsc.parallel_loop(0, _LO_BINS, step=_L, unroll=8)
        def _(i):
            hist[pl.ds(i, _L)] = zeros

        total = fz
        for c in range(_NSCCHUNK):
            b = c % 2
            if c + 1 < _NSCCHUNK:
                nb = (c + 1) % 2
                copies[nb] = pltpu.async_copy(
                    _chunk_src(bits_hbm, wid, c + 1), bufs[nb], sem.at[nb])
            copies[b].wait()
            cur = bufs[b]

            @plsc.parallel_loop(0, _SCCHUNK, step=_L, unroll=16, carry=fz)
            def sub(i, s):
                v = cur[i >> 9, pl.ds(i & 511, _L)]
                top = lax.shift_right_logical(v, 15)
                low = v & 0x7FFF
                m_b = top == bv
                plsc.addupdate_scatter(hist, [low], ones, mask=m_b)
                m_a = top > bv
                f = plsc.bitcast(v, jnp.float32)
                return s + jnp.where(m_a, f, fz)

            total = total + sub

        pltpu.sync_copy(hist, hist_hbm.at[wid])
        acc[...] = total
        pltpu.sync_copy(acc, sum_hbm.at[wid])

    return k(bits3, b_vec)


def _tc_final(histB, sums, sel, dice):
    def k(h_ref, s_ref, sel_ref, dice_ref, out_ref):
        B = sel_ref[0]
        count_above = sel_ref[1]
        r = _K - count_above

        h = jnp.sum(h_ref[...], axis=0).reshape(256, 128)
        r_iota = lax.broadcasted_iota(jnp.int32, (256, 128), 0)
        c_iota = lax.broadcasted_iota(jnp.int32, (256, 128), 1)
        j_idx = r_iota * 128 + c_iota

        def cnt_ge(m):
            return jnp.sum(jnp.where(j_idx >= m, h, 0))

        def body(_, carry):
            lo, hi = carry
            mid = lo + (hi - lo + 1) // 2
            big = cnt_ge(mid) >= r
            return jnp.where(big, mid, lo), jnp.where(big, hi, mid - 1)

        jstar, _ = lax.fori_loop(0, 16, body,
                                 (jnp.int32(0), jnp.int32(_LO_BINS - 1)))
        cnt_gt_in_b = cnt_ge(jstar + 1)

        vals = lax.bitcast_convert_type(B * _LO_BINS + j_idx, jnp.float32)
        sum_gt_in_b = jnp.sum(
            jnp.where(j_idx > jstar, vals * h.astype(jnp.float32), 0.0))
        t_val = lax.bitcast_convert_type(B * _LO_BINS + jstar, jnp.float32)

        sum_above = jnp.sum(s_ref[...])
        n_t = (_K - count_above - cnt_gt_in_b).astype(jnp.float32)
        focal_loss = (sum_above + sum_gt_in_b + t_val * n_t) / _K

        dice_loss = 1.0 - (2.0 * dice_ref[2] + _SMOOTH) / (
            dice_ref[0] + dice_ref[1] + _SMOOTH)
        out_ref[0] = _DICE_WEIGHT * dice_loss + (1.0 - _DICE_WEIGHT) * focal_loss

    return pl.pallas_call(
        k,
        in_specs=[
            pl.BlockSpec((_NW, _LO_BINS), lambda: (0, 0)),
            pl.BlockSpec((_NW, _L), lambda: (0, 0)),
            pl.BlockSpec(memory_space=pltpu.SMEM),
            pl.BlockSpec(memory_space=pltpu.SMEM),
        ],
        out_specs=pl.BlockSpec(memory_space=pltpu.SMEM),
        out_shape=jax.ShapeDtypeStruct((1,), jnp.float32),
    )(histB, sums, sel, dice)


def kernel(pred, target):
    bits, dice = _tc_focal(pred, target)
    histA = _sc_hist_a(bits)
    sel = _tc_select_bin(histA)
    b_vec = jnp.broadcast_to(sel[0:1], (_L,))
    histB, sums = _sc_hist_b(bits, b_vec)
    return _tc_final(histB, sums, sel, dice)[0]
